# Initial kernel scaffold; baseline (speedup 1.0000x reference)
import functools

import jax
import jax.numpy as jnp
from jax import lax
from jax.experimental import pallas as pl
from jax.experimental.pallas import tpu as pltpu
from jax.experimental.pallas import tpu_sc as plsc

_B, _L, _D, _H, _E, _K = 2, 2048, 1024, 2048, 8, 2
_N = _B * _L
_R = 256
_PR = _N * _K + _E * _R
_NBR = _PR // _R
_NSH = _N // _R
_P = _PR + _N
_NB = _NBR + _NSH
_NW = 32
_GCH = 64
_CCH = 32


def _gate_body(x_ref, g_ref, idx_ref, w_ref):
    x = x_ref[...]
    g = g_ref[...]
    xn = x / jnp.maximum(
        jnp.sqrt(jnp.sum(x * x, axis=1, keepdims=True)), 1e-12)
    gn = g / jnp.maximum(
        jnp.sqrt(jnp.sum(g * g, axis=1, keepdims=True)), 1e-12)
    logits = lax.dot_general(xn, gn, (((1,), (1,)), ((), ())),
                             preferred_element_type=jnp.float32)
    scores = jax.nn.sigmoid(logits)
    lane = lax.broadcasted_iota(jnp.int32, scores.shape, 1)
    m1 = jnp.max(scores, axis=1, keepdims=True)
    i1 = jnp.min(jnp.where(scores == m1, lane, _E), axis=1, keepdims=True)
    s2 = jnp.where(lane == i1, -jnp.inf, scores)
    m2 = jnp.max(s2, axis=1, keepdims=True)
    i2 = jnp.min(jnp.where(s2 == m2, lane, _E), axis=1, keepdims=True)
    tot = jnp.maximum(m1 + m2, 1e-12)
    idx_ref[...] = jnp.where(lane == 0, i1, jnp.where(lane == 1, i2, 0))
    w_ref[...] = jnp.where(lane == 0, m1 / tot,
                           jnp.where(lane == 1, m2 / tot, 0.0))


def _gate(xf, gate_W):
    return pl.pallas_call(
        _gate_body,
        out_shape=(jax.ShapeDtypeStruct((_N, _E), jnp.int32),
                   jax.ShapeDtypeStruct((_N, _E), jnp.float32)),
    )(xf, gate_W)


def _moe_body(eidx_ref, valid_ref, x_ref, w_ref, wg_ref, wu_ref, wd_ref,
              out_ref):
    b = pl.program_id(0)

    @pl.when(valid_ref[b] != 0)
    def _():
        xb = x_ref[...].astype(jnp.bfloat16)
        gg = lax.dot_general(xb, wg_ref[0], (((1,), (1,)), ((), ())),
                             preferred_element_type=jnp.float32)
        uu = lax.dot_general(xb, wu_ref[0], (((1,), (1,)), ((), ())),
                             preferred_element_type=jnp.float32)
        act = (gg * jax.nn.sigmoid(gg)) * uu
        y = lax.dot_general(act.astype(jnp.bfloat16), wd_ref[0],
                            (((1,), (1,)), ((), ())),
                            preferred_element_type=jnp.float32)
        out_ref[...] = y * w_ref[...]


def _grouped_swiglu(block_expert, valid, xs, w_col, Wg_all, Wu_all, Wd_all):
    grid_spec = pltpu.PrefetchScalarGridSpec(
        num_scalar_prefetch=2,
        grid=(_NB,),
        in_specs=[
            pl.BlockSpec((_R, _D), lambda b, e, v: (b, 0)),
            pl.BlockSpec((_R, 1), lambda b, e, v: (b, 0)),
            pl.BlockSpec((1, _H, _D), lambda b, e, v: (e[b], 0, 0)),
            pl.BlockSpec((1, _H, _D), lambda b, e, v: (e[b], 0, 0)),
            pl.BlockSpec((1, _D, _H), lambda b, e, v: (e[b], 0, 0)),
        ],
        out_specs=pl.BlockSpec((_R, _D), lambda b, e, v: (b, 0)),
    )
    return pl.pallas_call(
        _moe_body,
        grid_spec=grid_spec,
        out_shape=jax.ShapeDtypeStruct((_P, _D), jnp.float32),
    )(block_expert, valid, xs, w_col, Wg_all, Wu_all, Wd_all)


_sc_mesh = plsc.VectorSubcoreMesh(core_axis_name="c", subcore_axis_name="s")


@functools.partial(
    pl.kernel,
    mesh=_sc_mesh,
    out_type=jax.ShapeDtypeStruct((_P, _D), jnp.float32),
    scratch_types=[
        pltpu.VMEM((_GCH,), jnp.int32),
        pltpu.VMEM((_GCH, _D), jnp.float32),
        pltpu.SemaphoreType.DMA,
    ],
)
def _sc_gather(xf_hbm, src_hbm, out_hbm, idx_v, rows_v, sem):
    wid = lax.axis_index("s") * 2 + lax.axis_index("c")
    rows_per_w = _P // _NW
    base = wid * rows_per_w

    def chunk(ci, carry):
        off = pl.multiple_of(base + ci * _GCH, _GCH)
        pltpu.sync_copy(src_hbm.at[pl.ds(off, _GCH)], idx_v)
        pltpu.async_copy(xf_hbm.at[idx_v], rows_v, sem).wait()
        pltpu.sync_copy(rows_v, out_hbm.at[pl.ds(off, _GCH)])
        return carry

    lax.fori_loop(0, rows_per_w // _GCH, chunk, 0)


@functools.partial(
    pl.kernel,
    mesh=_sc_mesh,
    out_type=jax.ShapeDtypeStruct((_N, _D), jnp.float32),
    scratch_types=[
        pltpu.VMEM((_CCH,), jnp.int32),
        pltpu.VMEM((_CCH,), jnp.int32),
        pltpu.VMEM((_CCH, _D), jnp.float32),
        pltpu.VMEM((_CCH, _D), jnp.float32),
        pltpu.VMEM((_CCH, _D), jnp.float32),
        pltpu.SemaphoreType.DMA,
    ],
)
def _sc_combine(ys_hbm, pos0_hbm, pos1_hbm, out_hbm, i0_v, i1_v, a_v, b_v,
                c_v, sem):
    wid = lax.axis_index("s") * 2 + lax.axis_index("c")
    tok_per_w = _N // _NW
    base = wid * tok_per_w

    def chunk(ci, carry):
        t0 = pl.multiple_of(base + ci * _CCH, _CCH)
        pltpu.sync_copy(pos0_hbm.at[pl.ds(t0, _CCH)], i0_v)
        pltpu.sync_copy(pos1_hbm.at[pl.ds(t0, _CCH)], i1_v)
        cp0 = pltpu.async_copy(ys_hbm.at[i0_v], a_v, sem)
        cp1 = pltpu.async_copy(ys_hbm.at[i1_v], b_v, sem)
        pltpu.sync_copy(ys_hbm.at[pl.ds(_PR + t0, _CCH)], c_v)
        cp0.wait()
        cp1.wait()

        def row(r, rc):
            def col(cc, cci):
                s = pl.ds(cc * 16, 16)
                a_v[r, s] = a_v[r, s] + b_v[r, s] + c_v[r, s]
                return cci
            return lax.fori_loop(0, _D // 16, col, rc)

        lax.fori_loop(0, _CCH, row, 0)
        pltpu.sync_copy(a_v, out_hbm.at[pl.ds(t0, _CCH)])
        return carry

    lax.fori_loop(0, tok_per_w // _CCH, chunk, 0)


def _dispatch_plan(idx_pad, w_pad):
    ef = idx_pad[:, :_K].reshape(-1)
    wf = w_pad[:, :_K].reshape(-1)
    oh = (ef[:, None] == jnp.arange(_E, dtype=ef.dtype)).astype(jnp.int32)
    counts = jnp.sum(oh, axis=0)
    padded = ((counts + _R - 1) // _R) * _R
    start = jnp.concatenate(
        [jnp.zeros((1,), jnp.int32),
         jnp.cumsum(padded)[:-1].astype(jnp.int32)])
    rank = jnp.sum(jnp.cumsum(oh, axis=0) * oh, axis=1) - 1
    pos = (start[ef] + rank).astype(jnp.int32)
    tok = jnp.arange(_N * _K, dtype=jnp.int32) // _K
    src_r = jnp.zeros((_PR,), jnp.int32).at[pos].set(tok)
    w_r = jnp.zeros((_PR,), jnp.float32).at[pos].set(wf)
    src_full = jnp.concatenate([src_r, jnp.arange(_N, dtype=jnp.int32)])
    w_full = jnp.concatenate([w_r, jnp.ones((_N,), jnp.float32)])
    bb = jnp.arange(_NBR, dtype=jnp.int32) * _R
    be = jnp.searchsorted(start, bb, side="right").astype(jnp.int32) - 1
    valid_r = (bb < (start + counts)[be]).astype(jnp.int32)
    block_expert = jnp.concatenate(
        [jnp.clip(be, 0, _E - 1), jnp.full((_NSH,), _E, jnp.int32)])
    valid = jnp.concatenate([valid_r, jnp.ones((_NSH,), jnp.int32)])
    pos2 = pos.reshape(_N, _K)
    return src_full, w_full, block_expert, valid, pos2[:, 0], pos2[:, 1]


def kernel(x, gate_W, Wg, Wu, Wd, Wsg, Wsu, Wsd):
    xf = x.reshape(-1, _D)
    idx_pad, w_pad = _gate(xf, gate_W)
    src_full, w_full, block_expert, valid, pos0, pos1 = _dispatch_plan(
        idx_pad, w_pad)
    xs = _sc_gather(xf, src_full)
    Wg_all = jnp.concatenate([Wg, Wsg[None]], 0).astype(jnp.bfloat16)
    Wu_all = jnp.concatenate([Wu, Wsu[None]], 0).astype(jnp.bfloat16)
    Wd_all = jnp.concatenate([Wd, Wsd[None]], 0).astype(jnp.bfloat16)
    ys = _grouped_swiglu(block_expert, valid, xs, w_full[:, None],
                         Wg_all, Wu_all, Wd_all)
    out = _sc_combine(ys, pos0, pos1)
    return out.reshape(_B, _L, _D)


# speedup vs baseline: 1.0247x; 1.0247x over previous
import functools

import jax
import jax.numpy as jnp
from jax import lax
from jax.experimental import pallas as pl
from jax.experimental.pallas import tpu as pltpu
from jax.experimental.pallas import tpu_sc as plsc

_B, _L, _D, _H, _E, _K = 2, 2048, 1024, 2048, 8, 2
_N = _B * _L
_R = 256
_PR = _N * _K + _E * _R
_NBR = _PR // _R
_NSH = _N // _R
_P = _PR + _N
_NB = _NBR + _NSH
_NW = 32
_GCH = 64
_CCH = 32


def _gate_body(s_ref, idx_ref, w_ref):
    scores = s_ref[...]
    lane = lax.broadcasted_iota(jnp.int32, scores.shape, 1)
    m1 = jnp.max(scores, axis=1, keepdims=True)
    i1 = jnp.min(jnp.where(scores == m1, lane, _E), axis=1, keepdims=True)
    s2 = jnp.where(lane == i1, -jnp.inf, scores)
    m2 = jnp.max(s2, axis=1, keepdims=True)
    i2 = jnp.min(jnp.where(s2 == m2, lane, _E), axis=1, keepdims=True)
    tot = jnp.maximum(m1 + m2, 1e-12)
    idx_ref[...] = jnp.where(lane == 0, i1, jnp.where(lane == 1, i2, 0))
    w_ref[...] = jnp.where(lane == 0, m1 / tot,
                           jnp.where(lane == 1, m2 / tot, 0.0))


def _gate(scores):
    return pl.pallas_call(
        _gate_body,
        out_shape=(jax.ShapeDtypeStruct((_N, _E), jnp.int32),
                   jax.ShapeDtypeStruct((_N, _E), jnp.float32)),
    )(scores)


def _moe_body(eidx_ref, valid_ref, x_ref, w_ref, wg_ref, wu_ref, wd_ref,
              out_ref):
    b = pl.program_id(0)

    @pl.when(valid_ref[b] != 0)
    def _():
        xb = x_ref[...].astype(jnp.bfloat16)
        gg = lax.dot_general(xb, wg_ref[0], (((1,), (1,)), ((), ())),
                             preferred_element_type=jnp.float32)
        uu = lax.dot_general(xb, wu_ref[0], (((1,), (1,)), ((), ())),
                             preferred_element_type=jnp.float32)
        act = (gg * jax.nn.sigmoid(gg)) * uu
        y = lax.dot_general(act.astype(jnp.bfloat16), wd_ref[0],
                            (((1,), (1,)), ((), ())),
                            preferred_element_type=jnp.float32)
        out_ref[...] = y * w_ref[...]


def _grouped_swiglu(block_expert, valid, xs, w_col, Wg_all, Wu_all, Wd_all):
    grid_spec = pltpu.PrefetchScalarGridSpec(
        num_scalar_prefetch=2,
        grid=(_NB,),
        in_specs=[
            pl.BlockSpec((_R, _D), lambda b, e, v: (b, 0)),
            pl.BlockSpec((_R, 1), lambda b, e, v: (b, 0)),
            pl.BlockSpec((1, _H, _D), lambda b, e, v: (e[b], 0, 0)),
            pl.BlockSpec((1, _H, _D), lambda b, e, v: (e[b], 0, 0)),
            pl.BlockSpec((1, _D, _H), lambda b, e, v: (e[b], 0, 0)),
        ],
        out_specs=pl.BlockSpec((_R, _D), lambda b, e, v: (b, 0)),
    )
    return pl.pallas_call(
        _moe_body,
        grid_spec=grid_spec,
        out_shape=jax.ShapeDtypeStruct((_P, _D), jnp.float32),
    )(block_expert, valid, xs, w_col, Wg_all, Wu_all, Wd_all)


@functools.lru_cache(maxsize=None)
def _make_sc_gather():
    mesh = plsc.VectorSubcoreMesh(core_axis_name="c", subcore_axis_name="s")

    @functools.partial(
        pl.kernel,
        mesh=mesh,
        out_type=jax.ShapeDtypeStruct((_P, _D), jnp.float32),
        scratch_types=[
            pltpu.VMEM((_GCH,), jnp.int32),
            pltpu.VMEM((_GCH, _D), jnp.float32),
            pltpu.SemaphoreType.DMA,
        ],
    )
    def _sc_gather(xf_hbm, src_hbm, out_hbm, idx_v, rows_v, sem):
        wid = lax.axis_index("s") * 2 + lax.axis_index("c")
        rows_per_w = _P // _NW
        base = wid * rows_per_w

        def chunk(ci, carry):
            off = pl.multiple_of(base + ci * _GCH, _GCH)
            pltpu.sync_copy(src_hbm.at[pl.ds(off, _GCH)], idx_v)
            pltpu.async_copy(xf_hbm.at[idx_v], rows_v, sem).wait()
            pltpu.sync_copy(rows_v, out_hbm.at[pl.ds(off, _GCH)])
            return carry

        lax.fori_loop(0, rows_per_w // _GCH, chunk, 0)

    return _sc_gather


@functools.lru_cache(maxsize=None)
def _make_sc_combine():
    mesh = plsc.VectorSubcoreMesh(core_axis_name="c", subcore_axis_name="s")

    @functools.partial(
        pl.kernel,
        mesh=mesh,
        out_type=jax.ShapeDtypeStruct((_N, _D), jnp.float32),
        scratch_types=[
            pltpu.VMEM((_CCH,), jnp.int32),
            pltpu.VMEM((_CCH,), jnp.int32),
            pltpu.VMEM((_CCH, _D), jnp.float32),
            pltpu.VMEM((_CCH, _D), jnp.float32),
            pltpu.VMEM((_CCH, _D), jnp.float32),
            pltpu.SemaphoreType.DMA,
        ],
    )
    def _sc_combine(ys_hbm, pos0_hbm, pos1_hbm, out_hbm, i0_v, i1_v, a_v,
                    b_v, c_v, sem):
        wid = lax.axis_index("s") * 2 + lax.axis_index("c")
        tok_per_w = _N // _NW
        base = wid * tok_per_w

        def chunk(ci, carry):
            t0 = pl.multiple_of(base + ci * _CCH, _CCH)
            pltpu.sync_copy(pos0_hbm.at[pl.ds(t0, _CCH)], i0_v)
            pltpu.sync_copy(pos1_hbm.at[pl.ds(t0, _CCH)], i1_v)
            cp0 = pltpu.async_copy(ys_hbm.at[i0_v], a_v, sem)
            cp1 = pltpu.async_copy(ys_hbm.at[i1_v], b_v, sem)
            pltpu.sync_copy(ys_hbm.at[pl.ds(_PR + t0, _CCH)], c_v)
            cp0.wait()
            cp1.wait()

            def row(r, rc):
                def col(cc, cci):
                    s = pl.ds(cc * 16, 16)
                    a_v[r, s] = a_v[r, s] + b_v[r, s] + c_v[r, s]
                    return cci
                return lax.fori_loop(0, _D // 16, col, rc)

            lax.fori_loop(0, _CCH, row, 0)
            pltpu.sync_copy(a_v, out_hbm.at[pl.ds(t0, _CCH)])
            return carry

        lax.fori_loop(0, tok_per_w // _CCH, chunk, 0)

    return _sc_combine


def _dispatch_plan(idx_pad, w_pad):
    ef = idx_pad[:, :_K].reshape(-1)
    wf = w_pad[:, :_K].reshape(-1)
    oh = (ef[:, None] == jnp.arange(_E, dtype=ef.dtype)).astype(jnp.int32)
    counts = jnp.sum(oh, axis=0)
    padded = ((counts + _R - 1) // _R) * _R
    start = jnp.concatenate(
        [jnp.zeros((1,), jnp.int32),
         jnp.cumsum(padded)[:-1].astype(jnp.int32)])
    rank = jnp.sum(jnp.cumsum(oh, axis=0) * oh, axis=1) - 1
    pos = (start[ef] + rank).astype(jnp.int32)
    tok = jnp.arange(_N * _K, dtype=jnp.int32) // _K
    src_r = jnp.zeros((_PR,), jnp.int32).at[pos].set(tok)
    w_r = jnp.zeros((_PR,), jnp.float32).at[pos].set(wf)
    src_full = jnp.concatenate([src_r, jnp.arange(_N, dtype=jnp.int32)])
    w_full = jnp.concatenate([w_r, jnp.ones((_N,), jnp.float32)])
    bb = jnp.arange(_NBR, dtype=jnp.int32) * _R
    be = jnp.searchsorted(start, bb, side="right").astype(jnp.int32) - 1
    valid_r = (bb < (start + counts)[be]).astype(jnp.int32)
    block_expert = jnp.concatenate(
        [jnp.clip(be, 0, _E - 1), jnp.full((_NSH,), _E, jnp.int32)])
    valid = jnp.concatenate([valid_r, jnp.ones((_NSH,), jnp.int32)])
    pos2 = pos.reshape(_N, _K)
    return src_full, w_full, block_expert, valid, pos2[:, 0], pos2[:, 1]


def kernel(x, gate_W, Wg, Wu, Wd, Wsg, Wsu, Wsd):
    xf = x.reshape(-1, _D)
    xn = xf / jnp.maximum(jnp.linalg.norm(xf, axis=-1, keepdims=True), 1e-12)
    gwn = gate_W / jnp.maximum(
        jnp.linalg.norm(gate_W, axis=-1, keepdims=True), 1e-12)
    scores = jax.nn.sigmoid(xn @ gwn.T)
    idx_pad, w_pad = _gate(scores)
    src_full, w_full, block_expert, valid, pos0, pos1 = _dispatch_plan(
        idx_pad, w_pad)
    xs = _make_sc_gather()(xf, src_full)
    Wg_all = jnp.concatenate([Wg, Wsg[None]], 0).astype(jnp.bfloat16)
    Wu_all = jnp.concatenate([Wu, Wsu[None]], 0).astype(jnp.bfloat16)
    Wd_all = jnp.concatenate([Wd, Wsd[None]], 0).astype(jnp.bfloat16)
    ys = _grouped_swiglu(block_expert, valid, xs, w_full[:, None],
                         Wg_all, Wu_all, Wd_all)
    out = _make_sc_combine()(ys, pos0, pos1)
    return out.reshape(_B, _L, _D)
